# baseline (speedup 1.0000x reference)
import jax
import jax.numpy as jnp
from jax.experimental import pallas as pl
from jax.experimental.pallas import tpu as pltpu

S, D = 2048, 1024
P1, P2 = 256, 64


def _prep_body(x_ref, p_ref, xt_ref, xp1t_ref, xp2t_ref):
    xb = x_ref[...].astype(jnp.bfloat16)
    xt = xb.T
    xt_ref[...] = xt
    pb = p_ref[...].astype(jnp.bfloat16)
    xpt = jax.lax.dot_general(
        pb, xt, (((0,), (0,)), ((), ())),
        preferred_element_type=jnp.float32,
    ).astype(jnp.bfloat16)
    xp1t_ref[...] = xpt[:P1]
    xp2t_ref[...] = xpt[P1:]


def _tmatmul_body(wt_ref, xt_ref, b_ref, o_ref):
    wb = wt_ref[...].astype(jnp.bfloat16)
    acc = jnp.dot(wb, xt_ref[...], preferred_element_type=jnp.float32)
    o_ref[...] = acc + b_ref[...].T


def _tiled_tmatmul(wt, xt, bias_row, tn):
    n, k = wt.shape
    grid = pl.cdiv(n, tn)
    out_t = pl.pallas_call(
        _tmatmul_body,
        grid=(grid,),
        in_specs=[
            pl.BlockSpec((tn, k), lambda j: (j, 0)),
            pl.BlockSpec((k, S), lambda j: (0, 0)),
            pl.BlockSpec((1, tn), lambda j: (0, j)),
        ],
        out_specs=pl.BlockSpec((tn, S), lambda j: (j, 0)),
        out_shape=jax.ShapeDtypeStruct((n, S), jnp.float32),
        compiler_params=pltpu.CompilerParams(
            dimension_semantics=("parallel",),
        ),
    )(wt, xt, bias_row)
    return out_t.T.reshape(1, S, n)


def _transpose_cast_body(wt_ref, o_ref):
    o_ref[...] = wt_ref[...].astype(jnp.bfloat16).T


def _transpose_cast(wt, tn):
    n, k = wt.shape
    return pl.pallas_call(
        _transpose_cast_body,
        grid=(pl.cdiv(n, tn),),
        in_specs=[pl.BlockSpec((tn, k), lambda j: (j, 0))],
        out_specs=pl.BlockSpec((k, tn), lambda j: (0, j)),
        out_shape=jax.ShapeDtypeStruct((k, n), jnp.bfloat16),
        compiler_params=pltpu.CompilerParams(
            dimension_semantics=("parallel",),
        ),
    )(wt)


def _tmatmul_tlhs_body(w_ref, xt_ref, b_ref, o_ref):
    wb = w_ref[...].astype(jnp.bfloat16)
    acc = jax.lax.dot_general(
        wb, xt_ref[...], (((0,), (0,)), ((), ())),
        preferred_element_type=jnp.float32,
    )
    o_ref[...] = acc + b_ref[...].T


def _tiled_tmatmul_tlhs(w, xt, bias_row, tn):
    k, n = w.shape
    grid = pl.cdiv(n, tn)
    out_t = pl.pallas_call(
        _tmatmul_tlhs_body,
        grid=(grid,),
        in_specs=[
            pl.BlockSpec((k, tn), lambda j: (0, j)),
            pl.BlockSpec((k, S), lambda j: (0, 0)),
            pl.BlockSpec((1, tn), lambda j: (0, j)),
        ],
        out_specs=pl.BlockSpec((tn, S), lambda j: (j, 0)),
        out_shape=jax.ShapeDtypeStruct((n, S), jnp.float32),
        compiler_params=pltpu.CompilerParams(
            dimension_semantics=("parallel",),
        ),
    )(w, xt, bias_row)
    return out_t.T.reshape(1, S, n)


def kernel(x, proj1, proj2, W0, W1, W2, b0, b1, b2):
    x2 = x.reshape(S, D)
    projc = jnp.concatenate([proj1, proj2], axis=1)
    xt, xp1t, xp2t = pl.pallas_call(
        _prep_body,
        out_shape=(
            jax.ShapeDtypeStruct((D, S), jnp.bfloat16),
            jax.ShapeDtypeStruct((P1, S), jnp.bfloat16),
            jax.ShapeDtypeStruct((P2, S), jnp.bfloat16),
        ),
    )(x2, projc)

    w1b = _transpose_cast(W1.T, 4096)
    head = _tiled_tmatmul(W0.T, xt, b0.reshape(1, -1), 1024)
    c1 = _tiled_tmatmul_tlhs(w1b, xp1t, b1.reshape(1, -1), 2048)
    c2 = _tiled_tmatmul_tlhs(W2, xp2t, b2.reshape(1, -1), 2048)
    return (head, c1, c2)


# speedup vs baseline: 1.0177x; 1.0177x over previous
import jax
import jax.numpy as jnp
from jax.experimental import pallas as pl
from jax.experimental.pallas import tpu as pltpu

S, D = 2048, 1024
P1, P2 = 256, 64


def _prep_body(x_ref, p_ref, xt_ref, xp1t_ref, xp2t_ref):
    xb = x_ref[...].astype(jnp.bfloat16)
    xt = xb.T
    xt_ref[...] = xt
    pb = p_ref[...].astype(jnp.bfloat16)
    xpt = jax.lax.dot_general(
        pb, xt, (((0,), (0,)), ((), ())),
        preferred_element_type=jnp.float32,
    ).astype(jnp.bfloat16)
    xp1t_ref[...] = xpt[:P1]
    xp2t_ref[...] = xpt[P1:]


def _tmatmul_body(wt_ref, xt_ref, b_ref, o_ref):
    wb = wt_ref[...].astype(jnp.bfloat16)
    acc = jnp.dot(wb, xt_ref[...], preferred_element_type=jnp.float32)
    o_ref[...] = acc + b_ref[...].T


def _tiled_tmatmul(wt, xt, bias_row, tn):
    n, k = wt.shape
    grid = pl.cdiv(n, tn)
    out_t = pl.pallas_call(
        _tmatmul_body,
        grid=(grid,),
        in_specs=[
            pl.BlockSpec((tn, k), lambda j: (j, 0)),
            pl.BlockSpec((k, S), lambda j: (0, 0)),
            pl.BlockSpec((1, tn), lambda j: (0, j)),
        ],
        out_specs=pl.BlockSpec((tn, S), lambda j: (j, 0)),
        out_shape=jax.ShapeDtypeStruct((n, S), jnp.float32),
        compiler_params=pltpu.CompilerParams(
            dimension_semantics=("parallel",),
        ),
    )(wt, xt, bias_row)
    return out_t.T.reshape(1, S, n)


def _transpose_cast_body(wt_ref, o_ref):
    o_ref[...] = wt_ref[...].astype(jnp.bfloat16).T


def _transpose_cast(wt, tn):
    n, k = wt.shape
    return pl.pallas_call(
        _transpose_cast_body,
        grid=(pl.cdiv(n, tn),),
        in_specs=[pl.BlockSpec((tn, k), lambda j: (j, 0))],
        out_specs=pl.BlockSpec((k, tn), lambda j: (0, j)),
        out_shape=jax.ShapeDtypeStruct((k, n), jnp.bfloat16),
        compiler_params=pltpu.CompilerParams(
            dimension_semantics=("parallel",),
        ),
    )(wt)


def _tmatmul_tlhs_body(w_ref, xt_ref, b_ref, o_ref):
    wb = w_ref[...].astype(jnp.bfloat16)
    acc = jax.lax.dot_general(
        wb, xt_ref[...], (((0,), (0,)), ((), ())),
        preferred_element_type=jnp.float32,
    )
    o_ref[...] = acc + b_ref[...].T


def _tiled_tmatmul_tlhs(w, xt, bias_row, tn):
    k, n = w.shape
    grid = pl.cdiv(n, tn)
    out_t = pl.pallas_call(
        _tmatmul_tlhs_body,
        grid=(grid,),
        in_specs=[
            pl.BlockSpec((k, tn), lambda j: (0, j)),
            pl.BlockSpec((k, S), lambda j: (0, 0)),
            pl.BlockSpec((1, tn), lambda j: (0, j)),
        ],
        out_specs=pl.BlockSpec((tn, S), lambda j: (j, 0)),
        out_shape=jax.ShapeDtypeStruct((n, S), jnp.float32),
        compiler_params=pltpu.CompilerParams(
            dimension_semantics=("parallel",),
        ),
    )(w, xt, bias_row)
    return out_t.T.reshape(1, S, n)


def kernel(x, proj1, proj2, W0, W1, W2, b0, b1, b2):
    x2 = x.reshape(S, D)
    projc = jnp.concatenate([proj1, proj2], axis=1)
    xt, xp1t, xp2t = pl.pallas_call(
        _prep_body,
        out_shape=(
            jax.ShapeDtypeStruct((D, S), jnp.bfloat16),
            jax.ShapeDtypeStruct((P1, S), jnp.bfloat16),
            jax.ShapeDtypeStruct((P2, S), jnp.bfloat16),
        ),
    )(x2, projc)

    head = _tiled_tmatmul(W0.T, xt, b0.reshape(1, -1), 1024)
    c1 = _tiled_tmatmul(W1.T, xp1t, b1.reshape(1, -1), 2048)
    c2 = _tiled_tmatmul_tlhs(W2, xp2t, b2.reshape(1, -1), 2048)
    return (head, c1, c2)
